# Initial kernel scaffold; baseline (speedup 1.0000x reference)
import jax
import jax.numpy as jnp
from jax.experimental import pallas as pl


def kernel(x_a0, x_a1, x_b, ei_r0, ei_r1, ei_r2, ei_r3, ei_r4, ei_r5, batch_a0, batch_a1, batch_b, params):
    raise NotImplementedError("write your pallas kernel here")


# speedup vs baseline: 1.0149x; 1.0149x over previous
import functools

import jax
import jax.numpy as jnp
from jax import lax
from jax.experimental import pallas as pl
from jax.experimental.pallas import tpu as pltpu
from jax.experimental.pallas import tpu_sc as plsc

H = 128
DSUB = 32
ND = H // DSUB
NPA = 25600
NPB = 51200
AOFF = 25088
ACC_ROWS = 50176
ROWS_PT = ACC_ROWS // 16
EROWS_PT = 52
EPAD = 16 * EROWS_PT * 128
NBUF = 4
BN = 1600
NG = 64
PR = 144

@functools.lru_cache(maxsize=None)
def _sc_mesh():
    return plsc.VectorSubcoreMesh(
        core_axis_name="c", subcore_axis_name="s",
        num_cores=2, num_subcores=16)

_SRC_RELS = {"a_0": ["r1", "r4"], "a_1": ["r0", "r2", "r5"], "b": ["r3"]}
_DST_RELS = {"a_0": ["r5"], "a_1": ["r0", "r4"], "b": ["r1", "r2", "r3"]}
_K = {"a_0": 3, "a_1": 4, "b": 2}


def _mm_body_first(k, x_ref, w_ref, b_ref, *out_refs):
    x = x_ref[...]
    acc = jnp.dot(x, w_ref[...], preferred_element_type=jnp.float32,
                  precision=lax.Precision.HIGHEST)
    for j in range(k):
        for dd in range(ND):
            c0 = j * H + dd * DSUB
            out_refs[j * ND + dd][...] = (
                acc[:, c0:c0 + DSUB] + b_ref[j, dd * DSUB:(dd + 1) * DSUB][None, :])


def _mm_body_next(k, z0, z1, z2, z3, w_ref, b_ref, *out_refs):
    x = jnp.concatenate([z0[...], z1[...], z2[...], z3[...]], axis=1)
    x = jnp.maximum(x, 0.0)
    acc = jnp.dot(x, w_ref[...], preferred_element_type=jnp.float32,
                  precision=lax.Precision.HIGHEST)
    for j in range(k):
        for dd in range(ND):
            c0 = j * H + dd * DSUB
            out_refs[j * ND + dd][...] = (
                acc[:, c0:c0 + DSUB] + b_ref[j, dd * DSUB:(dd + 1) * DSUB][None, :])


@functools.partial(jax.jit, static_argnums=(3,))
def _tc_transform_first(x, w, b, k):
    n_pad = x.shape[0]
    grid = (n_pad // BN,)
    return pl.pallas_call(
        functools.partial(_mm_body_first, k),
        grid=grid,
        in_specs=[
            pl.BlockSpec((BN, H), lambda i: (i, 0)),
            pl.BlockSpec((H, H * k), lambda i: (0, 0)),
            pl.BlockSpec((k, H), lambda i: (0, 0)),
        ],
        out_specs=[pl.BlockSpec((BN, DSUB), lambda i: (i, 0))] * (k * ND),
        out_shape=[jax.ShapeDtypeStruct((n_pad, DSUB), jnp.float32)] * (k * ND),
    )(x, w, b)


@functools.partial(jax.jit, static_argnums=(6,))
def _tc_transform_next(z0, z1, z2, z3, w, b, k):
    n_pad = z0.shape[0]
    grid = (n_pad // BN,)
    return pl.pallas_call(
        functools.partial(_mm_body_next, k),
        grid=grid,
        in_specs=(
            [pl.BlockSpec((BN, DSUB), lambda i: (i, 0))] * 4
            + [pl.BlockSpec((H, H * k), lambda i: (0, 0)),
               pl.BlockSpec((k, H), lambda i: (0, 0))]),
        out_specs=[pl.BlockSpec((BN, DSUB), lambda i: (i, 0))] * (k * ND),
        out_shape=[jax.ShapeDtypeStruct((n_pad, DSUB), jnp.float32)] * (k * ND),
    )(z0, z1, z2, z3, w, b)


def _sc_layer_body(*refs):
    ya0 = refs[0:12]
    ya1 = refs[12:28]
    yb = refs[28:36]
    es = refs[36:42]
    ed = refs[42:48]
    za0 = refs[48:52]
    za1 = refs[52:56]
    zb = refs[56:60]
    acc = refs[60]
    rows = refs[61]
    sidx = refs[62]
    didx = refs[63]
    sems = refs[64:64 + NBUF]

    cid = lax.axis_index("c")
    sid = lax.axis_index("s")
    base = sid * ROWS_PT
    base_hi = jnp.maximum(sid - 8, 0) * ROWS_PT

    core0_rels = ((ya0, 0, es[1], ed[1]),
                  (ya1, 4, es[2], ed[2]),
                  (yb, 0, es[3], ed[3]))
    core1_rels = ((ya1, 0, es[0], ed[0]),
                  (ya0, 4, es[4], ed[4]),
                  (ya1, 8, es[5], ed[5]))

    for d in range(ND):
        @pl.when(cid == 0)
        def _(d=d):
            pltpu.sync_copy(yb[4 + d].at[pl.ds(base, ROWS_PT)],
                            acc.at[pl.ds(base, ROWS_PT)])

        @pl.when(jnp.logical_and(cid == 1, sid < 8))
        def _(d=d):
            pltpu.sync_copy(ya0[8 + d].at[pl.ds(base, ROWS_PT)],
                            acc.at[pl.ds(base, ROWS_PT)])

        @pl.when(jnp.logical_and(cid == 1, sid >= 8))
        def _(d=d):
            pltpu.sync_copy(ya1[12 + d].at[pl.ds(base_hi, ROWS_PT)],
                            acc.at[pl.ds(AOFF + base_hi, ROWS_PT)])

        plsc.subcore_barrier()

        for core, rels in ((0, core0_rels), (1, core1_rels)):
            @pl.when(cid == core)
            def _(rels=rels, d=d):
                for yt, jb, sh, dh in rels:
                    ysrc = yt[jb + d]
                    pltpu.sync_copy(sh.at[sid], sidx)
                    pltpu.sync_copy(dh.at[sid], didx)

                    def outer(oi, c, ysrc=ysrc):
                        o = oi * NBUF
                        descs = [
                            pltpu.async_copy(
                                ysrc.at[sidx.at[o + b2]],
                                rows.at[b2], sems[b2])
                            for b2 in range(NBUF)]
                        for b2 in range(NBUF):
                            descs[b2].wait()
                            pltpu.sync_copy(rows.at[b2],
                                            acc.at[didx.at[o + b2]],
                                            add=True)
                        return c

                    lax.fori_loop(0, EROWS_PT // NBUF, outer, 0)

        plsc.subcore_barrier()

        @pl.when(cid == 0)
        def _(d=d):
            pltpu.sync_copy(acc.at[pl.ds(base, ROWS_PT)],
                            zb[d].at[pl.ds(base, ROWS_PT)])

        @pl.when(jnp.logical_and(cid == 1, sid < 8))
        def _(d=d):
            pltpu.sync_copy(acc.at[pl.ds(base, ROWS_PT)],
                            za0[d].at[pl.ds(base, ROWS_PT)])

        @pl.when(jnp.logical_and(cid == 1, sid >= 8))
        def _(d=d):
            pltpu.sync_copy(acc.at[pl.ds(AOFF + base_hi, ROWS_PT)],
                            za1[d].at[pl.ds(base_hi, ROWS_PT)])


@functools.lru_cache(maxsize=None)
def _sc_layer():
    return pl.kernel(
        _sc_layer_body,
        out_type=[jax.ShapeDtypeStruct((NPA, DSUB), jnp.float32)] * 8
        + [jax.ShapeDtypeStruct((NPB, DSUB), jnp.float32)] * 4,
        mesh=_sc_mesh(),
        compiler_params=pltpu.CompilerParams(use_tc_tiling_on_sc=False),
        scratch_types=(
            [pltpu.VMEM_SHARED((ACC_ROWS, DSUB), jnp.float32),
             pltpu.VMEM((NBUF, 128, DSUB), jnp.float32)]
            + [pltpu.VMEM((EROWS_PT, 128), jnp.int32)] * 2
            + [pltpu.SemaphoreType.DMA] * NBUF),
    )


def _sc_pool_body(*refs):
    za0 = refs[0:4]
    za1 = refs[4:8]
    zb = refs[8:12]
    ba0, ba1, bb = refs[12], refs[13], refs[14]
    out_sums = refs[15]
    out_cnt = refs[16]
    accs = refs[17:21]
    cnt = refs[21]
    zbufs = refs[22:26]
    zz16 = refs[26]
    ones = refs[27]
    bidx = refs[28]

    cid = lax.axis_index("c")
    sid = lax.axis_index("s")

    def initbuf(i, c):
        ones[i, pl.ds(0, 16)] = jnp.full((16,), 1.0, jnp.float32)
        zz16[i, pl.ds(0, 16)] = jnp.zeros((16,), jnp.float32)
        zbufs[0][i, pl.ds(0, 16)] = jnp.zeros((16,), jnp.float32)
        zbufs[0][i, pl.ds(16, 16)] = jnp.zeros((16,), jnp.float32)
        return c
    lax.fori_loop(0, 128, initbuf, 0)

    @pl.when(sid == 0)
    def _():
        for a in accs:
            pltpu.sync_copy(zbufs[0], a.at[pl.ds(0, 128)])
            pltpu.sync_copy(zbufs[0], a.at[pl.ds(PR - 128, 128)])
        pltpu.sync_copy(zz16, cnt.at[pl.ds(0, 128)])
        pltpu.sync_copy(zz16, cnt.at[pl.ds(PR - 128, 128)])

    plsc.subcore_barrier()

    def run(zlist, b_hbm, tid):
        pltpu.sync_copy(b_hbm.at[tid], bidx)

        def blk(i, c):
            r0 = (tid * 25 + i) * 128
            for dd in range(4):
                pltpu.sync_copy(zlist[dd].at[pl.ds(r0, 128)], zbufs[dd])

            def rl(r, c2):
                for dd in range(4):
                    for cc in (0, 16):
                        v = zbufs[dd][r, pl.ds(cc, 16)]
                        zbufs[dd][r, pl.ds(cc, 16)] = jnp.maximum(v, 0.0)
                return c2
            lax.fori_loop(0, 128, rl, 0)
            for dd in range(4):
                pltpu.sync_copy(zbufs[dd], accs[dd].at[bidx.at[i]], add=True)
            pltpu.sync_copy(ones, cnt.at[bidx.at[i]], add=True)
            return c
        lax.fori_loop(0, 25, blk, 0)

    @pl.when(cid == 0)
    def _():
        run(zb, bb, sid)

    @pl.when(jnp.logical_and(cid == 1, sid < 8))
    def _():
        run(za0, ba0, jnp.minimum(sid, 7))

    @pl.when(jnp.logical_and(cid == 1, sid >= 8))
    def _():
        run(za1, ba1, jnp.maximum(sid - 8, 0))

    plsc.subcore_barrier()

    for dd in range(4):
        @pl.when(sid == dd)
        def _(dd=dd):
            pltpu.sync_copy(accs[dd], out_sums.at[cid * 4 + dd])

    @pl.when(sid == 4)
    def _():
        pltpu.sync_copy(cnt, out_cnt.at[cid])


@functools.lru_cache(maxsize=None)
def _sc_pool():
    return pl.kernel(
        _sc_pool_body,
        out_type=[jax.ShapeDtypeStruct((8, PR, DSUB), jnp.float32),
                  jax.ShapeDtypeStruct((2, PR, 16), jnp.float32)],
        mesh=_sc_mesh(),
        compiler_params=pltpu.CompilerParams(use_tc_tiling_on_sc=False),
        scratch_types=(
            [pltpu.VMEM_SHARED((PR, DSUB), jnp.float32)] * 4
            + [pltpu.VMEM_SHARED((PR, 16), jnp.float32)]
            + [pltpu.VMEM((128, DSUB), jnp.float32)] * 4
            + [pltpu.VMEM((128, 16), jnp.float32)] * 2
            + [pltpu.VMEM((25, 128), jnp.int32)]),
    )


def _mlp_body(s_ref, c_ref, w1, b1, w2, b2, w3, b3, wo, bo, out_ref):
    s = s_ref[...]
    c = c_ref[...]

    def pool(blk0, row0, cnt_core):
        cols = [s[blk0 + d, row0:row0 + NG, :] for d in range(ND)]
        m = jnp.concatenate(cols, axis=1)
        n = c[cnt_core, row0:row0 + NG, 0:1]
        return m / jnp.maximum(n, 1.0)

    h = jnp.concatenate([pool(4, 0, 1), pool(4, 72, 1), pool(0, 0, 0)], axis=1)
    h = jnp.maximum(jnp.dot(h, w1[...], preferred_element_type=jnp.float32,
                            precision=lax.Precision.HIGHEST)
                    + b1[...], 0.0)
    h = jnp.maximum(jnp.dot(h, w2[...], preferred_element_type=jnp.float32,
                            precision=lax.Precision.HIGHEST)
                    + b2[...], 0.0)
    h = jnp.maximum(jnp.dot(h, w3[...], preferred_element_type=jnp.float32,
                            precision=lax.Precision.HIGHEST)
                    + b3[...], 0.0)
    out_ref[...] = jnp.dot(h, wo[...], preferred_element_type=jnp.float32,
                         precision=lax.Precision.HIGHEST) + bo[...]


def _tc_mlp(sums, cnt, p):
    return pl.pallas_call(
        _mlp_body,
        out_shape=jax.ShapeDtypeStruct((NG, 1), jnp.float32),
    )(sums, cnt,
      p["lin1_W"], p["lin1_b"].reshape(1, -1),
      p["lin2_W"], p["lin2_b"].reshape(1, -1),
      p["lin3_W"], p["lin3_b"].reshape(1, -1),
      p["out_W"], p["out_b"].reshape(1, -1))


def _pad_rows(x, n_pad):
    return jnp.pad(x, ((0, n_pad - x.shape[0]), (0, 0)))


def _prep_edges(ei, dst_shift, dst_trash):
    src = ei[0].astype(jnp.int32)
    dst = ei[1].astype(jnp.int32) + dst_shift
    e = src.shape[0]
    per_tile = e // 16
    cap = EROWS_PT * 128
    src = jnp.pad(src.reshape(16, per_tile), ((0, 0), (0, cap - per_tile)))
    dst = jnp.pad(dst.reshape(16, per_tile), ((0, 0), (0, cap - per_tile)),
                  constant_values=dst_trash)
    return (src.reshape(16, EROWS_PT, 128), dst.reshape(16, EROWS_PT, 128))


def _prep_batch(b, n_pad, shift, trash):
    b = b.astype(jnp.int32) + shift
    b = jnp.pad(b, (0, n_pad - b.shape[0]), constant_values=trash)
    return b.reshape(n_pad // 3200, 25, 128)


def kernel(x_a0, x_a1, x_b, ei_r0, ei_r1, ei_r2, ei_r3, ei_r4, ei_r5,
           batch_a0, batch_a1, batch_b, params):
    n = {"a_0": x_a0.shape[0], "a_1": x_a1.shape[0], "b": x_b.shape[0]}

    s0, d0 = _prep_edges(ei_r0, AOFF, AOFF + n["a_1"])
    s1, d1 = _prep_edges(ei_r1, 0, n["b"])
    s2, d2 = _prep_edges(ei_r2, 0, n["b"])
    s3, d3 = _prep_edges(ei_r3, 0, n["b"])
    s4, d4 = _prep_edges(ei_r4, AOFF, AOFF + n["a_1"])
    s5, d5 = _prep_edges(ei_r5, 0, n["a_0"])
    edges = (s0, s1, s2, s3, s4, s5, d0, d1, d2, d3, d4, d5)

    ws, bs = {}, {}
    for t in ("a_0", "a_1", "b"):
        ws[t], bs[t] = [], []
        for lay in params["convs"]:
            blocks = [lay[r]["Wr"] for r in _SRC_RELS[t]]
            blocks.append(sum(lay[r]["Wroot"] for r in _DST_RELS[t]))
            ws[t].append(jnp.concatenate(blocks, axis=1))
            bias = jnp.zeros((_K[t], H), jnp.float32)
            bias = bias.at[-1].set(sum(lay[r]["br"] for r in _DST_RELS[t]))
            bs[t].append(bias)

    xp = {"a_0": _pad_rows(x_a0, NPA), "a_1": _pad_rows(x_a1, NPA),
          "b": _pad_rows(x_b, NPB)}

    z = None
    for li in range(len(params["convs"])):
        ys = {}
        for t in ("a_0", "a_1", "b"):
            if li == 0:
                ys[t] = _tc_transform_first(xp[t], ws[t][li], bs[t][li], _K[t])
            else:
                zt = z[t]
                ys[t] = _tc_transform_next(zt[0], zt[1], zt[2], zt[3],
                                           ws[t][li], bs[t][li], _K[t])
        outs = _sc_layer()(*ys["a_0"], *ys["a_1"], *ys["b"], *edges)
        z = {"a_0": outs[0:4], "a_1": outs[4:8], "b": outs[8:12]}

    bp0 = _prep_batch(batch_a0, NPA, 0, NG)
    bp1 = _prep_batch(batch_a1, NPA, 72, 72 + NG)
    bpb = _prep_batch(batch_b, NPB, 0, NG)
    sums, cnt = _sc_pool()(*z["a_0"], *z["a_1"], *z["b"], bp0, bp1, bpb)

    out = _tc_mlp(sums, cnt, params)
    return jnp.reshape(out, (-1,))


# speedup vs baseline: 1.5399x; 1.5173x over previous
import functools

import jax
import jax.numpy as jnp
from jax import lax
from jax.experimental import pallas as pl
from jax.experimental.pallas import tpu as pltpu
from jax.experimental.pallas import tpu_sc as plsc

H = 128
DSUB = 32
ND = H // DSUB
NPA = 25600
NPB = 51200
AOFF = 25088
ACC_ROWS = 50176
ROWS_PT = ACC_ROWS // 16
EROWS_PT = 52
EPAD = 16 * EROWS_PT * 128
NBUF = 4
NOUT = EROWS_PT // NBUF
BN = 1600
NG = 64
PR = 144


@functools.lru_cache(maxsize=None)
def _sc_mesh():
    return plsc.VectorSubcoreMesh(
        core_axis_name="c", subcore_axis_name="s",
        num_cores=2, num_subcores=16)

_SRC_RELS = {"a_0": ["r1", "r4"], "a_1": ["r0", "r2", "r5"], "b": ["r3"]}
_DST_RELS = {"a_0": ["r5"], "a_1": ["r0", "r4"], "b": ["r1", "r2", "r3"]}
_K = {"a_0": 3, "a_1": 4, "b": 2}


def _mm_body(k, relu, x_ref, w_ref, b_ref, msg_ref, root_ref):
    x = x_ref[...]
    if relu:
        x = jnp.maximum(x, 0.0)
    acc = jnp.dot(x, w_ref[...], preferred_element_type=jnp.float32,
                  precision=lax.Precision.HIGHEST)
    msg_ref[...] = acc[:, :H * (k - 1)]
    root_ref[...] = acc[:, H * (k - 1):] + b_ref[...]


@functools.partial(jax.jit, static_argnums=(3, 4))
def _tc_transform(x, w, b, k, relu):
    n_pad = x.shape[0]
    grid = (n_pad // BN,)
    return pl.pallas_call(
        functools.partial(_mm_body, k, relu),
        grid=grid,
        in_specs=[
            pl.BlockSpec((BN, H), lambda i: (i, 0)),
            pl.BlockSpec((H, H * k), lambda i: (0, 0)),
            pl.BlockSpec((1, H), lambda i: (0, 0)),
        ],
        out_specs=[pl.BlockSpec((BN, H * (k - 1)), lambda i: (i, 0)),
                   pl.BlockSpec((BN, H), lambda i: (i, 0))],
        out_shape=[jax.ShapeDtypeStruct((n_pad, H * (k - 1)), jnp.float32),
                   jax.ShapeDtypeStruct((n_pad, H), jnp.float32)],
    )(x, w, b)


def _sc_layer_body(*refs):
    (msg_a0, msg_a1, msg_b, root_a0, root_a1, root_b) = refs[0:6]
    es = refs[6:12]
    ed = refs[12:18]
    za0, za1, zb = refs[18:21]
    acc = refs[21]
    rows = refs[22]
    sidx = refs[23]
    didx = refs[24]
    gsems = refs[25:25 + NBUF]
    ssems = refs[25 + NBUF:25 + 2 * NBUF]

    cid = lax.axis_index("c")
    sid = lax.axis_index("s")
    base = sid * ROWS_PT
    base_hi = jnp.maximum(sid - 8, 0) * ROWS_PT

    core0_rels = ((msg_a0, es[1], ed[1]),
                  (msg_a1, es[2], ed[2]),
                  (msg_b, es[3], ed[3]))
    core1_rels = ((msg_a1, es[0], ed[0]),
                  (msg_a0, es[4], ed[4]),
                  (msg_a1, es[5], ed[5]))

    for d in range(ND):
        @pl.when(cid == 0)
        def _(d=d):
            pltpu.sync_copy(
                root_b.at[pl.ds(base, ROWS_PT), pl.ds(d * DSUB, DSUB)],
                acc.at[pl.ds(base, ROWS_PT)])

        @pl.when(jnp.logical_and(cid == 1, sid < 8))
        def _(d=d):
            pltpu.sync_copy(
                root_a0.at[pl.ds(base, ROWS_PT), pl.ds(d * DSUB, DSUB)],
                acc.at[pl.ds(base, ROWS_PT)])

        @pl.when(jnp.logical_and(cid == 1, sid >= 8))
        def _(d=d):
            pltpu.sync_copy(
                root_a1.at[pl.ds(base_hi, ROWS_PT), pl.ds(d * DSUB, DSUB)],
                acc.at[pl.ds(AOFF + base_hi, ROWS_PT)])

        plsc.subcore_barrier()

        for core, rels in ((0, core0_rels), (1, core1_rels)):
            @pl.when(cid == core)
            def _(rels=rels, d=d):
                for mt, sh, dh in rels:
                    ysrc = mt.at[pl.ds(d, mt.shape[0] - (ND - 1))]
                    pltpu.sync_copy(sh.at[sid], sidx)
                    pltpu.sync_copy(dh.at[sid], didx)

                    def fire(blk, b2, ysrc=ysrc):
                        return pltpu.async_copy(
                            ysrc.at[sidx.at[blk]], rows.at[b2], gsems[b2])

                    for b2 in range(NBUF):
                        fire(b2, b2)

                    def outer(oi, c, ysrc=ysrc):
                        o = oi * NBUF
                        for b2 in range(NBUF):
                            pltpu.make_async_copy(
                                ysrc.at[sidx.at[o + b2]], rows.at[b2],
                                gsems[b2]).wait()
                            pltpu.async_copy(rows.at[b2],
                                             acc.at[didx.at[o + b2]],
                                             ssems[b2], add=True)
                        for b2 in range(NBUF):
                            pltpu.make_async_copy(
                                rows.at[b2], acc.at[didx.at[o + b2]],
                                ssems[b2]).wait()

                            @pl.when(oi < NOUT - 1)
                            def _(b2=b2, o=o, ysrc=ysrc):
                                fire(o + NBUF + b2, b2)
                        return c

                    lax.fori_loop(0, NOUT, outer, 0)

        plsc.subcore_barrier()

        @pl.when(cid == 0)
        def _(d=d):
            pltpu.sync_copy(
                acc.at[pl.ds(base, ROWS_PT)],
                zb.at[pl.ds(base, ROWS_PT), pl.ds(d * DSUB, DSUB)])

        @pl.when(jnp.logical_and(cid == 1, sid < 8))
        def _(d=d):
            pltpu.sync_copy(
                acc.at[pl.ds(base, ROWS_PT)],
                za0.at[pl.ds(base, ROWS_PT), pl.ds(d * DSUB, DSUB)])

        @pl.when(jnp.logical_and(cid == 1, sid >= 8))
        def _(d=d):
            pltpu.sync_copy(
                acc.at[pl.ds(AOFF + base_hi, ROWS_PT)],
                za1.at[pl.ds(base_hi, ROWS_PT), pl.ds(d * DSUB, DSUB)])


@functools.lru_cache(maxsize=None)
def _sc_layer():
    return pl.kernel(
        _sc_layer_body,
        out_type=[jax.ShapeDtypeStruct((NPA, H), jnp.float32),
                  jax.ShapeDtypeStruct((NPA, H), jnp.float32),
                  jax.ShapeDtypeStruct((NPB, H), jnp.float32)],
        mesh=_sc_mesh(),
        compiler_params=pltpu.CompilerParams(use_tc_tiling_on_sc=False),
        scratch_types=(
            [pltpu.VMEM_SHARED((ACC_ROWS, DSUB), jnp.float32),
             pltpu.VMEM((NBUF, 128, DSUB), jnp.float32)]
            + [pltpu.VMEM((EROWS_PT, 128), jnp.int32)] * 2
            + [pltpu.SemaphoreType.DMA] * (2 * NBUF)),
    )


def _sc_pool_body(*refs):
    za0, za1, zb = refs[0:3]
    ba0, ba1, bb = refs[3:6]
    out_sums = refs[6]
    out_cnt = refs[7]
    acc = refs[8]
    cnt = refs[9]
    zbuf = refs[10]
    zz16 = refs[11]
    ones = refs[12]
    bidx = refs[13]

    cid = lax.axis_index("c")
    sid = lax.axis_index("s")

    def initbuf(i, c):
        ones[i, pl.ds(0, 16)] = jnp.full((16,), 1.0, jnp.float32)
        zz16[i, pl.ds(0, 16)] = jnp.zeros((16,), jnp.float32)
        for cc in range(8):
            zbuf[i, pl.ds(cc * 16, 16)] = jnp.zeros((16,), jnp.float32)
        return c
    lax.fori_loop(0, 128, initbuf, 0)

    @pl.when(sid == 0)
    def _():
        pltpu.sync_copy(zbuf, acc.at[pl.ds(0, 128)])
        pltpu.sync_copy(zbuf, acc.at[pl.ds(PR - 128, 128)])
        pltpu.sync_copy(zz16, cnt.at[pl.ds(0, 128)])
        pltpu.sync_copy(zz16, cnt.at[pl.ds(PR - 128, 128)])

    plsc.subcore_barrier()

    def run(z, b_hbm, tid):
        pltpu.sync_copy(b_hbm.at[tid], bidx)

        def blk(i, c):
            r0 = (tid * 25 + i) * 128
            pltpu.sync_copy(z.at[pl.ds(r0, 128)], zbuf)

            def rl(r, c2):
                for cc in range(8):
                    v = zbuf[r, pl.ds(cc * 16, 16)]
                    zbuf[r, pl.ds(cc * 16, 16)] = jnp.maximum(v, 0.0)
                return c2
            lax.fori_loop(0, 128, rl, 0)
            pltpu.sync_copy(zbuf, acc.at[bidx.at[i]], add=True)
            pltpu.sync_copy(ones, cnt.at[bidx.at[i]], add=True)
            return c
        lax.fori_loop(0, 25, blk, 0)

    @pl.when(cid == 0)
    def _():
        run(zb, bb, sid)

    @pl.when(jnp.logical_and(cid == 1, sid < 8))
    def _():
        run(za0, ba0, jnp.minimum(sid, 7))

    @pl.when(jnp.logical_and(cid == 1, sid >= 8))
    def _():
        run(za1, ba1, jnp.maximum(sid - 8, 0))

    plsc.subcore_barrier()

    @pl.when(sid == 0)
    def _():
        pltpu.sync_copy(acc, out_sums.at[cid])

    @pl.when(sid == 1)
    def _():
        pltpu.sync_copy(cnt, out_cnt.at[cid])


@functools.lru_cache(maxsize=None)
def _sc_pool():
    return pl.kernel(
        _sc_pool_body,
        out_type=[jax.ShapeDtypeStruct((2, PR, H), jnp.float32),
                  jax.ShapeDtypeStruct((2, PR, 16), jnp.float32)],
        mesh=_sc_mesh(),
        compiler_params=pltpu.CompilerParams(use_tc_tiling_on_sc=False),
        scratch_types=(
            [pltpu.VMEM_SHARED((PR, H), jnp.float32),
             pltpu.VMEM_SHARED((PR, 16), jnp.float32),
             pltpu.VMEM((128, H), jnp.float32)]
            + [pltpu.VMEM((128, 16), jnp.float32)] * 2
            + [pltpu.VMEM((25, 128), jnp.int32)]),
    )


def _mlp_body(s_ref, c_ref, w1, b1, w2, b2, w3, b3, wo, bo, out_ref):
    s = s_ref[...]
    c = c_ref[...]

    def pool(core, row0):
        m = s[core, row0:row0 + NG, :]
        n = c[core, row0:row0 + NG, 0:1]
        return m / jnp.maximum(n, 1.0)

    h = jnp.concatenate([pool(1, 0), pool(1, 72), pool(0, 0)], axis=1)
    h = jnp.maximum(jnp.dot(h, w1[...], preferred_element_type=jnp.float32,
                            precision=lax.Precision.HIGHEST) + b1[...], 0.0)
    h = jnp.maximum(jnp.dot(h, w2[...], preferred_element_type=jnp.float32,
                            precision=lax.Precision.HIGHEST) + b2[...], 0.0)
    h = jnp.maximum(jnp.dot(h, w3[...], preferred_element_type=jnp.float32,
                            precision=lax.Precision.HIGHEST) + b3[...], 0.0)
    out_ref[...] = jnp.dot(h, wo[...], preferred_element_type=jnp.float32,
                           precision=lax.Precision.HIGHEST) + bo[...]


def _tc_mlp(sums, cnt, p):
    return pl.pallas_call(
        _mlp_body,
        out_shape=jax.ShapeDtypeStruct((NG, 1), jnp.float32),
    )(sums, cnt,
      p["lin1_W"], p["lin1_b"].reshape(1, -1),
      p["lin2_W"], p["lin2_b"].reshape(1, -1),
      p["lin3_W"], p["lin3_b"].reshape(1, -1),
      p["out_W"], p["out_b"].reshape(1, -1))


def _pad_rows(x, n_pad):
    return jnp.pad(x, ((0, n_pad - x.shape[0]), (0, 0)))


def _prep_edges(ei, mult, j, dst_shift, dst_trash):
    src = ei[0].astype(jnp.int32) * mult + 4 * j
    dst = ei[1].astype(jnp.int32) + dst_shift
    e = src.shape[0]
    per_tile = e // 16
    cap = EROWS_PT * 128
    src = jnp.pad(src.reshape(16, per_tile), ((0, 0), (0, cap - per_tile)))
    dst = jnp.pad(dst.reshape(16, per_tile), ((0, 0), (0, cap - per_tile)),
                  constant_values=dst_trash)
    return (src.reshape(16, EROWS_PT, 128), dst.reshape(16, EROWS_PT, 128))


def _prep_batch(b, n_pad, shift, trash):
    b = b.astype(jnp.int32) + shift
    b = jnp.pad(b, (0, n_pad - b.shape[0]), constant_values=trash)
    return b.reshape(n_pad // 3200, 25, 128)


def kernel(x_a0, x_a1, x_b, ei_r0, ei_r1, ei_r2, ei_r3, ei_r4, ei_r5,
           batch_a0, batch_a1, batch_b, params):
    n = {"a_0": x_a0.shape[0], "a_1": x_a1.shape[0], "b": x_b.shape[0]}
    m = {t: 4 * (_K[t] - 1) for t in _K}

    s0, d0 = _prep_edges(ei_r0, m["a_1"], 0, AOFF, AOFF + n["a_1"])
    s1, d1 = _prep_edges(ei_r1, m["a_0"], 0, 0, n["b"])
    s2, d2 = _prep_edges(ei_r2, m["a_1"], 1, 0, n["b"])
    s3, d3 = _prep_edges(ei_r3, m["b"], 0, 0, n["b"])
    s4, d4 = _prep_edges(ei_r4, m["a_0"], 1, AOFF, AOFF + n["a_1"])
    s5, d5 = _prep_edges(ei_r5, m["a_1"], 2, 0, n["a_0"])
    edges = (s0, s1, s2, s3, s4, s5, d0, d1, d2, d3, d4, d5)

    ws, bs = {}, {}
    for t in ("a_0", "a_1", "b"):
        ws[t], bs[t] = [], []
        for lay in params["convs"]:
            blocks = [lay[r]["Wr"] for r in _SRC_RELS[t]]
            blocks.append(sum(lay[r]["Wroot"] for r in _DST_RELS[t]))
            ws[t].append(jnp.concatenate(blocks, axis=1))
            bs[t].append(sum(lay[r]["br"] for r in _DST_RELS[t]).reshape(1, H))

    z = {"a_0": _pad_rows(x_a0, NPA), "a_1": _pad_rows(x_a1, NPA),
         "b": _pad_rows(x_b, NPB)}

    for li in range(len(params["convs"])):
        msgs, roots = {}, {}
        for t in ("a_0", "a_1", "b"):
            mm, rr = _tc_transform(z[t], ws[t][li], bs[t][li], _K[t], li > 0)
            msgs[t] = jnp.reshape(mm, (-1, DSUB))
            roots[t] = rr
        za0, za1, zbv = _sc_layer()(
            msgs["a_0"], msgs["a_1"], msgs["b"],
            roots["a_0"], roots["a_1"], roots["b"], *edges)
        z = {"a_0": za0, "a_1": za1, "b": zbv}

    bp0 = _prep_batch(batch_a0, NPA, 0, NG)
    bp1 = _prep_batch(batch_a1, NPA, 72, 72 + NG)
    bpb = _prep_batch(batch_b, NPB, 0, NG)
    sums, cnt = _sc_pool()(z["a_0"], z["a_1"], z["b"], bp0, bp1, bpb)

    out = _tc_mlp(sums, cnt, params)
    return jnp.reshape(out, (-1,))
